# baseline (speedup 1.0000x reference)
import functools

import jax
import jax.numpy as jnp
from jax import lax
from jax.experimental import pallas as pl
from jax.experimental.pallas import tpu as pltpu
from jax.experimental.pallas import tpu_sc as plsc

_NSPLIT = 2


def _moe_kernel(nsplit, x_ref, wg_ref, bg_ref, be_ref, *rest):
    we_refs = rest[:nsplit]
    out_ref = rest[nsplit]
    s_ref, cw_ref = rest[nsplit + 1:]
    t = pl.program_id(0)

    @pl.when(t == 0)
    def _():
        x = x_ref[...]
        E = wg_ref.shape[0]
        ii = lax.broadcasted_iota(jnp.int32, (E, E), 0)
        jj = lax.broadcasted_iota(jnp.int32, (E, E), 1)
        eye = (ii == jj).astype(jnp.float32)
        bg_col = lax.dot_general(
            eye, bg_ref[...], (((1,), (1,)), ((), ())),
            preferred_element_type=jnp.float32)
        logits = lax.dot_general(
            wg_ref[...], x, (((1,), (1,)), ((), ())),
            preferred_element_type=jnp.float32) + bg_col
        eids = lax.broadcasted_iota(jnp.int32, logits.shape, 0)
        v1 = jnp.max(logits, axis=0, keepdims=True)
        i1 = jnp.min(jnp.where(logits == v1, eids, E), axis=0, keepdims=True)
        oh1 = eids == i1
        masked = jnp.where(oh1, -jnp.inf, logits)
        v2 = jnp.max(masked, axis=0, keepdims=True)
        i2 = jnp.min(jnp.where(masked == v2, eids, E), axis=0, keepdims=True)
        oh2 = eids == i2
        p = jnp.exp(v2 - v1)
        w1 = 1.0 / (1.0 + p)
        w2 = p / (1.0 + p)
        coef = w1 * oh1.astype(jnp.float32) + w2 * oh2.astype(jnp.float32)
        s_ref[...] = lax.dot_general(
            coef, x, (((1,), (0,)), ((), ())),
            preferred_element_type=jnp.float32)
        cw_ref[...] = jnp.sum(coef, axis=1, keepdims=True)

    contrib = lax.dot_general(
        s_ref[pl.ds(t * nsplit, 1), :], we_refs[0][0],
        (((1,), (1,)), ((), ())),
        preferred_element_type=jnp.float32)
    for j in range(1, nsplit):
        contrib = contrib + lax.dot_general(
            s_ref[pl.ds(t * nsplit + j, 1), :], we_refs[j][0],
            (((1,), (1,)), ((), ())),
            preferred_element_type=jnp.float32)

    @pl.when(t == 0)
    def _():
        bias = jnp.sum(cw_ref[...] * be_ref[...], axis=0, keepdims=True)
        out_ref[...] = contrib + bias

    @pl.when(t != 0)
    def _():
        out_ref[...] = out_ref[...] + contrib


def _bcast_kernel(t_ref, s_ref, out_ref):
    out_ref[...] = jnp.broadcast_to(t_ref[...] + s_ref[...], out_ref.shape)


def _make_sc_stream(O, D):
    mesh = plsc.VectorSubcoreMesh(core_axis_name="c", subcore_axis_name="s")

    @functools.partial(
        pl.kernel, mesh=mesh,
        out_type=jax.ShapeDtypeStruct((O,), jnp.float32),
        scratch_types=[
            pltpu.VMEM((2, 32, D), jnp.float32),
            pltpu.VMEM((32,), jnp.float32),
        ],
    )
    def sck(we_hbm, out_hbm, rows_v, out_v):
        c = lax.axis_index("c")
        s = lax.axis_index("s")
        wid = s * 2 + c
        obase = wid * 32
        for e in range(2):
            pltpu.sync_copy(we_hbm.at[6 + e, pl.ds(obase, 32)], rows_v.at[e])
        chunk = rows_v[0, 0, pl.ds(0, 16)] * 1e-30
        out_v[pl.ds(0, 16)] = chunk
        out_v[pl.ds(16, 16)] = chunk
        pltpu.sync_copy(out_v, out_hbm.at[pl.ds(obase, 32)])

    return sck


def kernel(x, Wg, bg, We, be):
    B, D = x.shape
    E, O, _ = We.shape
    ns = _NSPLIT
    we_specs = [
        pl.BlockSpec((1, O, D), functools.partial(
            lambda t, j: (t * ns + j, 0, 0), j=j))
        for j in range(ns)
    ]
    srow = _make_sc_stream(O, D)(We)
    tot = pl.pallas_call(
        functools.partial(_moe_kernel, ns),
        grid=(E // ns,),
        in_specs=[
            pl.BlockSpec((B, D), lambda t: (0, 0)),
            pl.BlockSpec((E, D), lambda t: (0, 0)),
            pl.BlockSpec((1, E), lambda t: (0, 0)),
            pl.BlockSpec((E, O), lambda t: (0, 0)),
        ] + we_specs,
        out_specs=pl.BlockSpec((1, O), lambda t: (0, 0)),
        out_shape=jax.ShapeDtypeStruct((1, O), jnp.float32),
        scratch_shapes=[
            pltpu.VMEM((E, D), jnp.float32),
            pltpu.VMEM((E, 1), jnp.float32),
        ],
    )(x, Wg, bg.reshape(1, E), be, *([We] * ns))
    out = pl.pallas_call(
        _bcast_kernel,
        out_shape=jax.ShapeDtypeStruct((B, O), jnp.float32),
    )(tot, srow.reshape(1, O))
    return out.astype(x.dtype)


# speedup vs baseline: 1.8466x; 1.8466x over previous
import functools

import jax
import jax.numpy as jnp
from jax.experimental import pallas as pl
from jax.experimental.pallas import tpu as pltpu

_NSPLIT = 2


def _moe_kernel(nsplit, x_ref, wg_ref, bg_ref, be_ref, *rest):
    we_refs = rest[:nsplit]
    out_ref = rest[nsplit]
    s_ref, cw_ref, tot_ref = rest[nsplit + 1:]
    t = pl.program_id(0)
    nexp = pl.num_programs(0)

    @pl.when(t == 0)
    def _():
        x = x_ref[...]
        E = wg_ref.shape[0]
        ii = jax.lax.broadcasted_iota(jnp.int32, (E, E), 0)
        jj = jax.lax.broadcasted_iota(jnp.int32, (E, E), 1)
        eye = (ii == jj).astype(jnp.float32)
        bg_col = jax.lax.dot_general(
            eye, bg_ref[...], (((1,), (1,)), ((), ())),
            preferred_element_type=jnp.float32)
        logits = jax.lax.dot_general(
            wg_ref[...], x, (((1,), (1,)), ((), ())),
            preferred_element_type=jnp.float32) + bg_col
        eids = jax.lax.broadcasted_iota(jnp.int32, logits.shape, 0)
        v1 = jnp.max(logits, axis=0, keepdims=True)
        i1 = jnp.min(jnp.where(logits == v1, eids, E), axis=0, keepdims=True)
        oh1 = eids == i1
        masked = jnp.where(oh1, -jnp.inf, logits)
        v2 = jnp.max(masked, axis=0, keepdims=True)
        i2 = jnp.min(jnp.where(masked == v2, eids, E), axis=0, keepdims=True)
        oh2 = eids == i2
        p = jnp.exp(v2 - v1)
        w1 = 1.0 / (1.0 + p)
        w2 = p / (1.0 + p)
        coef = w1 * oh1.astype(jnp.float32) + w2 * oh2.astype(jnp.float32)
        s_ref[...] = jax.lax.dot_general(
            coef, x, (((1,), (0,)), ((), ())),
            preferred_element_type=jnp.float32)
        cw_ref[...] = jnp.sum(coef, axis=1, keepdims=True)

    contrib = jax.lax.dot_general(
        s_ref[pl.ds(t * nsplit, 1), :], we_refs[0][0],
        (((1,), (1,)), ((), ())),
        preferred_element_type=jnp.float32)
    for j in range(1, nsplit):
        contrib = contrib + jax.lax.dot_general(
            s_ref[pl.ds(t * nsplit + j, 1), :], we_refs[j][0],
            (((1,), (1,)), ((), ())),
            preferred_element_type=jnp.float32)

    @pl.when(t == 0)
    def _():
        bias = jnp.sum(cw_ref[...] * be_ref[...], axis=0,
                       keepdims=True)
        tot_ref[...] = contrib + bias

    @pl.when(t != 0)
    def _():
        tot_ref[...] = tot_ref[...] + contrib

    @pl.when(t == nexp - 1)
    def _():
        out_ref[...] = jnp.broadcast_to(tot_ref[...], out_ref.shape)


def kernel(x, Wg, bg, We, be):
    B, D = x.shape
    E, O, _ = We.shape
    ns = _NSPLIT
    we_specs = [
        pl.BlockSpec((1, O, D), functools.partial(
            lambda t, j: (t * ns + j, 0, 0), j=j))
        for j in range(ns)
    ]
    total = pl.pallas_call(
        functools.partial(_moe_kernel, ns),
        grid=(E // ns,),
        in_specs=[
            pl.BlockSpec((B, D), lambda t: (0, 0)),
            pl.BlockSpec((E, D), lambda t: (0, 0)),
            pl.BlockSpec((1, E), lambda t: (0, 0)),
            pl.BlockSpec((E, O), lambda t: (0, 0)),
        ] + we_specs,
        out_specs=pl.BlockSpec((B, O), lambda t: (0, 0)),
        out_shape=jax.ShapeDtypeStruct((B, O), jnp.float32),
        scratch_shapes=[
            pltpu.VMEM((E, D), jnp.float32),
            pltpu.VMEM((E, 1), jnp.float32),
            pltpu.VMEM((1, O), jnp.float32),
        ],
    )(x, Wg, bg.reshape(1, E), be, *([We] * ns))
    return total.astype(x.dtype)
